# baseline (speedup 1.0000x reference)
import functools

import jax
import jax.numpy as jnp
from jax import lax
from jax.experimental import pallas as pl
from jax.experimental.pallas import tpu as pltpu

_NEG = -3.0e38


def _ce_body(nc, v, tgt_ref, x_ref, ce_ref, m_ref, s_ref, t_ref):
    j = pl.program_id(1)
    x = x_ref[...]
    r, w = x.shape
    col = j * w + lax.broadcasted_iota(jnp.int32, (r, w), 1)
    xm = jnp.where(col < v, x, _NEG)
    eq = col == tgt_ref[...]
    contrib = jnp.sum(jnp.where(eq, x, 0.0), axis=1, keepdims=True)
    lm = jnp.max(xm, axis=1, keepdims=True)

    @pl.when(j == 0)
    def _():
        m_ref[...] = lm
        s_ref[...] = jnp.sum(jnp.exp(xm - lm), axis=1, keepdims=True)
        t_ref[...] = contrib

    @pl.when(j > 0)
    def _():
        m_old = m_ref[...]
        m_new = jnp.maximum(m_old, lm)
        s_ref[...] = s_ref[...] * jnp.exp(m_old - m_new) + jnp.sum(
            jnp.exp(xm - m_new), axis=1, keepdims=True)
        m_ref[...] = m_new
        t_ref[...] = t_ref[...] + contrib

    @pl.when(j == nc - 1)
    def _():
        ce_ref[...] = m_ref[...] + jnp.log(s_ref[...]) - t_ref[...]


def _monotone_key(bits):
    return bits ^ ((bits >> 31) & jnp.int32(0x7FFFFFFF))


def _topk_body(k_top, ce_ref, out_ref):
    ce = ce_ref[...]
    key = _monotone_key(lax.bitcast_convert_type(ce, jnp.int32))

    def body(_, lohi):
        lo, hi = lohi
        mid = (lo >> 1) + (hi >> 1) + ((lo | hi) & 1)
        cnt = jnp.sum((key >= mid).astype(jnp.int32))
        pred = cnt >= k_top
        return jnp.where(pred, mid, lo), jnp.where(pred, hi, mid - 1)

    lo0 = jnp.int32(-2147483647 - 1)
    hi0 = jnp.int32(2147483647)
    theta, _ = lax.fori_loop(0, 33, body, (lo0, hi0))
    kth_val = lax.bitcast_convert_type(_monotone_key(theta), jnp.float32)
    gt = key > theta
    cnt_gt = jnp.sum(gt.astype(jnp.int32))
    sum_gt = jnp.sum(jnp.where(gt, ce, 0.0))
    res = (sum_gt + (k_top - cnt_gt).astype(jnp.float32) * kth_val
           ) / jnp.float32(k_top)
    out_ref[...] = jnp.broadcast_to(res, (1, 1))


def kernel(logits, targets):
    b, v = logits.shape
    r = min(b, 256)
    w = min(v, 4096)
    nr = b // r
    nc = pl.cdiv(v, w)
    tgt2 = targets.astype(jnp.int32)[:, None]

    ce = pl.pallas_call(
        functools.partial(_ce_body, nc, v),
        grid=(nr, nc),
        in_specs=[
            pl.BlockSpec((r, 1), lambda i, j: (i, 0)),
            pl.BlockSpec((r, w), lambda i, j: (i, j)),
        ],
        out_specs=pl.BlockSpec((r, 1), lambda i, j: (i, 0)),
        out_shape=jax.ShapeDtypeStruct((b, 1), jnp.float32),
        scratch_shapes=[pltpu.VMEM((r, 1), jnp.float32)] * 3,
        compiler_params=pltpu.CompilerParams(
            dimension_semantics=("parallel", "arbitrary")),
    )(tgt2, logits)

    k_top = max(1, int(0.3 * b))
    ce_2d = ce.reshape(8, b // 8)
    out = pl.pallas_call(
        functools.partial(_topk_body, k_top),
        out_shape=jax.ShapeDtypeStruct((1, 1), jnp.float32),
    )(ce_2d)
    return out[0, 0]
